# Initial kernel scaffold; baseline (speedup 1.0000x reference)
import jax
import jax.numpy as jnp
from jax.experimental import pallas as pl


def kernel(edge_index, edge_label_index, ue, ie, uWl, uWr, ub, ug, ubb, iWl, iWr, ib, ig, ibb, fcW1, fcb1, fcW2, fcb2):
    raise NotImplementedError("write your pallas kernel here")


# speedup vs baseline: 2.3885x; 2.3885x over previous
import functools

import jax
import jax.numpy as jnp
from jax import lax
from jax.experimental import pallas as pl
from jax.experimental.pallas import tpu as pltpu
from jax.experimental.pallas import tpu_sc as plsc

N = 50000
H = 64
E = 800000
B = 200000
NLAYERS = 2
EPS = 1e-5

HALF = 25000
TRASH = HALF
HPAD = 25088
NPAD = 2 * HPAD
PADW = HPAD - HALF

NC = 2
NS = 16
ES = E // NS
G = 128
NG_FULL = ES // G
TAIL = ES - NG_FULL * G
ZS = HPAD // NS

_mesh = plsc.VectorSubcoreMesh(core_axis_name="c", subcore_axis_name="s")


def _remap_src(idx_ref, n16):
    for i in range(n16):
        v = idx_ref[pl.ds(i * 16, 16)]
        idx_ref[pl.ds(i * 16, 16)] = jnp.where(v >= HALF, v + PADW, v)


def _remap_dst(raw_ref, out_ref, c, n16):
    base = c * HALF
    for i in range(n16):
        v = raw_ref[pl.ds(i * 16, 16)] - base
        ok = (v >= 0) & (v < HALF)
        out_ref[pl.ds(i * 16, 16)] = jnp.where(ok, v, TRASH)


def _zero_and_barrier(zeros_hbm, acc_sh, s):
    pltpu.sync_copy(zeros_hbm, acc_sh.at[pl.ds(s * ZS, ZS)])
    plsc.subcore_barrier()


def _dump_half(acc_sh, out_hbm, c, s):
    pltpu.sync_copy(acc_sh.at[pl.ds(s * ZS, ZS)],
                    out_hbm.at[c, pl.ds(s * ZS, ZS)])


def _agg_body(src_hbm, dst_hbm, table_hbm, zeros_hbm, out_hbm,
              acc_sh, sidx, dtmp, ldst, sidx_t, ldst_t, rows, gsem):
    c = lax.axis_index("c")
    s = lax.axis_index("s")
    _zero_and_barrier(zeros_hbm, acc_sh, s)

    def do_group(base, gcount, si_ref, ld_ref):
        pltpu.sync_copy(src_hbm.at[pl.ds(base, gcount)], si_ref)
        pltpu.sync_copy(dst_hbm.at[pl.ds(base, gcount)],
                        dtmp.at[0, pl.ds(0, gcount)])
        _remap_src(si_ref, gcount // 16)
        _remap_dst(dtmp.at[0], ld_ref, c, gcount // 16)
        pltpu.async_copy(table_hbm.at[si_ref],
                         rows.at[pl.ds(0, gcount)], gsem).wait()
        pltpu.sync_copy(rows.at[pl.ds(0, gcount)],
                        acc_sh.at[ld_ref], add=True)

    def body(g, carry):
        do_group(s * ES + g * G, G, sidx.at[0], ldst.at[0])
        return carry

    lax.fori_loop(0, NG_FULL, body, 0)
    do_group(s * ES + NG_FULL * G, TAIL, sidx_t.at[0], ldst_t.at[0])
    plsc.subcore_barrier()
    _dump_half(acc_sh, out_hbm, c, s)


_SC_PARAMS = pltpu.CompilerParams(use_tc_tiling_on_sc=False)

_agg = functools.partial(
    pl.kernel,
    out_type=jax.ShapeDtypeStruct((NC, HPAD, H), jnp.float32),
    mesh=_mesh,
    compiler_params=_SC_PARAMS,
    scratch_types=[
        pltpu.VMEM_SHARED((HPAD, H), jnp.float32),
        pltpu.VMEM((1, G), jnp.int32),
        pltpu.VMEM((1, G), jnp.int32),
        pltpu.VMEM((1, G), jnp.int32),
        pltpu.VMEM((1, TAIL), jnp.int32),
        pltpu.VMEM((1, TAIL), jnp.int32),
        pltpu.VMEM((G, H), jnp.float32),
        pltpu.SemaphoreType.DMA,
    ],
)(_agg_body)


def _cnt_body(udst_hbm, idst_hbm, ones_hbm, zeros_hbm, outu_hbm, outi_hbm,
              acc_sh, dtmp, ldst, ldst_t, ones_v):
    c = lax.axis_index("c")
    s = lax.axis_index("s")
    pltpu.sync_copy(ones_hbm, ones_v)

    for dst_hbm, out_hbm in ((udst_hbm, outu_hbm), (idst_hbm, outi_hbm)):
        _zero_and_barrier(zeros_hbm, acc_sh, s)

        def do_group(base, gcount, ld_ref):
            pltpu.sync_copy(dst_hbm.at[pl.ds(base, gcount)],
                            dtmp.at[0, pl.ds(0, gcount)])
            _remap_dst(dtmp.at[0], ld_ref, c, gcount // 16)
            pltpu.sync_copy(ones_v.at[pl.ds(0, gcount)],
                            acc_sh.at[ld_ref], add=True)

        def body(g, carry):
            do_group(s * ES + g * G, G, ldst.at[0])
            return carry

        lax.fori_loop(0, NG_FULL, body, 0)
        do_group(s * ES + NG_FULL * G, TAIL, ldst_t.at[0])
        plsc.subcore_barrier()
        _dump_half(acc_sh, out_hbm, c, s)
        plsc.subcore_barrier()


_cnt = functools.partial(
    pl.kernel,
    out_type=(jax.ShapeDtypeStruct((NC, HPAD, H), jnp.float32),
              jax.ShapeDtypeStruct((NC, HPAD, H), jnp.float32)),
    mesh=_mesh,
    compiler_params=_SC_PARAMS,
    scratch_types=[
        pltpu.VMEM_SHARED((HPAD, H), jnp.float32),
        pltpu.VMEM((1, G), jnp.int32),
        pltpu.VMEM((1, G), jnp.int32),
        pltpu.VMEM((1, TAIL), jnp.int32),
        pltpu.VMEM((G, H), jnp.float32),
    ],
)(_cnt_body)


GB_FULL = B // G
GTAIL = B - GB_FULL * G
GPW = GB_FULL // 32
GREM = GB_FULL - GPW * 32


def _fgather_body(ui_hbm, ii_hbm, utab_hbm, itab_hbm, uh_hbm, ih_hbm,
                  eidx, rows, gsem):
    c = lax.axis_index("c")
    s = lax.axis_index("s")
    w = s * NC + c

    def do_pair(base, gcount, n16):
        for idx_hbm, tab_hbm, out_hbm in ((ui_hbm, utab_hbm, uh_hbm),
                                          (ii_hbm, itab_hbm, ih_hbm)):
            pltpu.sync_copy(idx_hbm.at[pl.ds(base, gcount)],
                            eidx.at[0, pl.ds(0, gcount)])
            _remap_src(eidx.at[0], n16)
            pltpu.async_copy(tab_hbm.at[eidx.at[0, pl.ds(0, gcount)]],
                             rows.at[pl.ds(0, gcount)], gsem).wait()
            pltpu.sync_copy(rows.at[pl.ds(0, gcount)],
                            out_hbm.at[pl.ds(base, gcount)])

    def body(k, carry):
        gid = k * 32 + w

        @pl.when(gid < GB_FULL)
        def _():
            do_pair(gid * G, G, G // 16)

        return carry

    lax.fori_loop(0, GPW + 1, body, 0)

    @pl.when(w == 31)
    def _():
        do_pair(GB_FULL * G, GTAIL, GTAIL // 16)


_fgather = functools.partial(
    pl.kernel,
    out_type=(jax.ShapeDtypeStruct((B, H), jnp.float32),
              jax.ShapeDtypeStruct((B, H), jnp.float32)),
    mesh=_mesh,
    compiler_params=_SC_PARAMS,
    scratch_types=[
        pltpu.VMEM((1, G), jnp.int32),
        pltpu.VMEM((G, H), jnp.float32),
        pltpu.SemaphoreType.DMA,
    ],
)(_fgather_body)


DBLK = 512
MBLK = 1000


def _dense_kernel(acc_ref, cnt_ref, x_ref, a_ref, c_ref, d_ref, o_ref):
    mean = acc_ref[...] / jnp.maximum(cnt_ref[...], 1.0)
    y = (jnp.dot(mean, a_ref[...], preferred_element_type=jnp.float32)
         + jnp.dot(x_ref[...], c_ref[...], preferred_element_type=jnp.float32)
         + d_ref[...])
    o_ref[...] = jnp.maximum(y, 0.0)


def _dense(acc, cnt, x, a, cm, d):
    grid = (NPAD // DBLK,)
    blk = pl.BlockSpec((DBLK, H), lambda i: (i, 0))
    wblk = pl.BlockSpec((H, H), lambda i: (0, 0))
    return pl.pallas_call(
        _dense_kernel,
        grid=grid,
        in_specs=[blk, blk, blk, wblk, wblk, pl.BlockSpec((1, H), lambda i: (0, 0))],
        out_specs=blk,
        out_shape=jax.ShapeDtypeStruct((NPAD, H), jnp.float32),
    )(acc, cnt, x, a, cm, d)


def _mlp_kernel(u_ref, i_ref, w1u_ref, w1i_ref, b1_ref, w2_ref, b2_ref, o_ref):
    h = (jnp.dot(u_ref[...], w1u_ref[...], preferred_element_type=jnp.float32)
         + jnp.dot(i_ref[...], w1i_ref[...], preferred_element_type=jnp.float32)
         + b1_ref[...])
    h = jnp.maximum(h, 0.0)
    o_ref[...] = (jnp.dot(h, w2_ref[...], preferred_element_type=jnp.float32)
                  + b2_ref[...])


def _mlp(uh, ih, w1u, w1i, b1, w2t, b2):
    grid = (B // MBLK,)
    blk = pl.BlockSpec((MBLK, H), lambda i: (i, 0))
    return pl.pallas_call(
        _mlp_kernel,
        grid=grid,
        in_specs=[blk, blk,
                  pl.BlockSpec((H, H), lambda i: (0, 0)),
                  pl.BlockSpec((H, H), lambda i: (0, 0)),
                  pl.BlockSpec((1, H), lambda i: (0, 0)),
                  pl.BlockSpec((H, 8), lambda i: (0, 0)),
                  pl.BlockSpec((1, 8), lambda i: (0, 0))],
        out_specs=pl.BlockSpec((MBLK, 8), lambda i: (i, 0)),
        out_shape=jax.ShapeDtypeStruct((B, 8), jnp.float32),
    )(uh, ih, w1u, w1i, b1, w2t, b2)


def _pad_table(x):
    z = jnp.zeros((PADW, H), jnp.float32)
    return jnp.concatenate([x[:HALF], z, x[HALF:], z], axis=0)


def kernel(edge_index, edge_label_index, ue, ie, uWl, uWr, ub, ug, ubb,
           iWl, iWr, ib, ig, ibb, fcW1, fcb1, fcW2, fcb2):
    u_idx = edge_index[0].astype(jnp.int32)
    i_idx = edge_index[1].astype(jnp.int32)
    eli_u = edge_label_index[0].astype(jnp.int32)
    eli_i = edge_label_index[1].astype(jnp.int32)

    zeros = jnp.zeros((ZS, H), jnp.float32)
    ones = jnp.ones((G, H), jnp.float32)

    upad = _pad_table(ue)
    ipad = _pad_table(ie)

    cntu, cnti = _cnt(u_idx, i_idx, ones, zeros)
    cntu = cntu.reshape(NPAD, H)
    cnti = cnti.reshape(NPAD, H)

    gscale = 1.0 / jnp.sqrt(1.0 + EPS)
    for l in range(NLAYERS):
        accu = _agg(i_idx, u_idx, ipad, zeros).reshape(NPAD, H)
        acci = _agg(u_idx, i_idx, upad, zeros).reshape(NPAD, H)
        gu = ug[l] * gscale
        gi = ig[l] * gscale
        au = uWl[l].T * gu[None, :]
        cu = uWr[l].T * gu[None, :]
        du = (ub[l] * gu + ubb[l])[None, :]
        ai = iWl[l].T * gi[None, :]
        ci = iWr[l].T * gi[None, :]
        di = (ib[l] * gi + ibb[l])[None, :]
        new_u = _dense(accu, cntu, upad, au, cu, du)
        new_i = _dense(acci, cnti, ipad, ai, ci, di)
        upad, ipad = new_u, new_i

    uh, ih = _fgather(eli_u, eli_i, upad, ipad)

    w1u = fcW1[:, :H].T
    w1i = fcW1[:, H:].T
    b1 = fcb1[None, :]
    w2t = jnp.zeros((H, 8), jnp.float32).at[:, :4].set(fcW2.T)
    b2 = jnp.zeros((1, 8), jnp.float32).at[0, :4].set(fcb2)
    out8 = _mlp(uh, ih, w1u, w1i, b1, w2t, b2)
    return out8[:, :4]


# speedup vs baseline: 3.2805x; 1.3735x over previous
import functools

import jax
import jax.numpy as jnp
from jax import lax
from jax.experimental import pallas as pl
from jax.experimental.pallas import tpu as pltpu
from jax.experimental.pallas import tpu_sc as plsc

N = 50000
H = 64
E = 800000
B = 200000
NLAYERS = 2
EPS = 1e-5

HALF = 25000
TRASH = HALF
HPAD = 25088
NPAD = 2 * HPAD
PADW = HPAD - HALF

NC = 2
NS = 16
ES = E // NS
G = 128
NG_FULL = ES // G
TAIL = ES - NG_FULL * G
ZS = HPAD // NS

_mesh = plsc.VectorSubcoreMesh(core_axis_name="c", subcore_axis_name="s")


def _remap_src(idx_ref, n16):
    for i in range(n16):
        v = idx_ref[pl.ds(i * 16, 16)]
        idx_ref[pl.ds(i * 16, 16)] = jnp.where(v >= HALF, v + PADW, v)


def _remap_dst(raw_ref, out_ref, c, n16):
    base = c * HALF
    for i in range(n16):
        v = raw_ref[pl.ds(i * 16, 16)] - base
        ok = (v >= 0) & (v < HALF)
        out_ref[pl.ds(i * 16, 16)] = jnp.where(ok, v, TRASH)


def _zero_and_barrier(zeros_hbm, acc_sh, s):
    pltpu.sync_copy(zeros_hbm, acc_sh.at[pl.ds(s * ZS, ZS)])
    plsc.subcore_barrier()


def _dump_half(acc_sh, out_hbm, c, s):
    pltpu.sync_copy(acc_sh.at[pl.ds(s * ZS, ZS)],
                    out_hbm.at[c, pl.ds(s * ZS, ZS)])


def _agg_body(src_hbm, dst_hbm, table_hbm, zeros_hbm, out_hbm,
              acc_sh, sidx, dtmp, ldst, sidx_t, ldst_t, rows,
              gsem0, gsem1, ssem0, ssem1):
    c = lax.axis_index("c")
    s = lax.axis_index("s")
    _zero_and_barrier(zeros_hbm, acc_sh, s)
    gsems = (gsem0, gsem1)
    ssems = (ssem0, ssem1)

    def load_group(g, b):
        base = s * ES + g * G
        pltpu.sync_copy(src_hbm.at[pl.ds(base, G)], sidx.at[b])
        pltpu.sync_copy(dst_hbm.at[pl.ds(base, G)], dtmp.at[0])
        _remap_src(sidx.at[b], G // 16)
        _remap_dst(dtmp.at[0], ldst.at[b], c, G // 16)

    def gather_start(b):
        pltpu.async_copy(table_hbm.at[sidx.at[b]], rows.at[b], gsems[b])

    def gather_wait(b):
        pltpu.make_async_copy(table_hbm.at[sidx.at[b]], rows.at[b],
                              gsems[b]).wait()

    def scatter_start(b):
        pltpu.async_copy(rows.at[b], acc_sh.at[ldst.at[b]], ssems[b],
                         add=True)

    def scatter_wait(b):
        pltpu.make_async_copy(rows.at[b], acc_sh.at[ldst.at[b]],
                              ssems[b]).wait()

    def body(g0, carry):
        for b in (0, 1):
            g = g0 * 2 + b

            @pl.when(g0 >= 1)
            def _():
                scatter_wait(b)

            load_group(g, b)
            gather_start(b)

            if b == 0:
                @pl.when(g0 >= 1)
                def _():
                    gather_wait(1)
                    scatter_start(1)
            else:
                gather_wait(0)
                scatter_start(0)
        return carry

    lax.fori_loop(0, NG_FULL // 2, body, 0)
    gather_wait(1)
    scatter_start(1)
    scatter_wait(0)
    base = s * ES + NG_FULL * G
    pltpu.sync_copy(src_hbm.at[pl.ds(base, TAIL)], sidx_t.at[0])
    pltpu.sync_copy(dst_hbm.at[pl.ds(base, TAIL)], dtmp.at[0, pl.ds(0, TAIL)])
    _remap_src(sidx_t.at[0], TAIL // 16)
    _remap_dst(dtmp.at[0], ldst_t.at[0], c, TAIL // 16)
    pltpu.async_copy(table_hbm.at[sidx_t.at[0]],
                     rows.at[0, pl.ds(0, TAIL)], gsem0).wait()
    scatter_wait(1)
    pltpu.sync_copy(rows.at[0, pl.ds(0, TAIL)],
                    acc_sh.at[ldst_t.at[0]], add=True)
    plsc.subcore_barrier()
    _dump_half(acc_sh, out_hbm, c, s)


_SC_PARAMS = pltpu.CompilerParams(use_tc_tiling_on_sc=False)

_agg = functools.partial(
    pl.kernel,
    out_type=jax.ShapeDtypeStruct((NC, HPAD, H), jnp.float32),
    mesh=_mesh,
    compiler_params=_SC_PARAMS,
    scratch_types=[
        pltpu.VMEM_SHARED((HPAD, H), jnp.float32),
        pltpu.VMEM((2, G), jnp.int32),
        pltpu.VMEM((1, G), jnp.int32),
        pltpu.VMEM((2, G), jnp.int32),
        pltpu.VMEM((1, TAIL), jnp.int32),
        pltpu.VMEM((1, TAIL), jnp.int32),
        pltpu.VMEM((2, G, H), jnp.float32),
        pltpu.SemaphoreType.DMA,
        pltpu.SemaphoreType.DMA,
        pltpu.SemaphoreType.DMA,
        pltpu.SemaphoreType.DMA,
    ],
)(_agg_body)


def _cnt_body(udst_hbm, idst_hbm, ones_hbm, zeros_hbm, outu_hbm, outi_hbm,
              acc_sh, dtmp, ldst, ldst_t, ones_v, ssem0, ssem1):
    c = lax.axis_index("c")
    s = lax.axis_index("s")
    pltpu.sync_copy(ones_hbm, ones_v)
    ssems = (ssem0, ssem1)

    for dst_hbm, out_hbm in ((udst_hbm, outu_hbm), (idst_hbm, outi_hbm)):
        _zero_and_barrier(zeros_hbm, acc_sh, s)

        def scatter_wait(b):
            pltpu.make_async_copy(ones_v, acc_sh.at[ldst.at[b]],
                                  ssems[b]).wait()

        def body(g0, carry):
            for b in (0, 1):
                g = g0 * 2 + b

                @pl.when(g0 >= 1)
                def _():
                    scatter_wait(b)

                base = s * ES + g * G
                pltpu.sync_copy(dst_hbm.at[pl.ds(base, G)],
                                dtmp.at[0])
                _remap_dst(dtmp.at[0], ldst.at[b], c, G // 16)
                pltpu.async_copy(ones_v, acc_sh.at[ldst.at[b]], ssems[b],
                                 add=True)
            return carry

        lax.fori_loop(0, NG_FULL // 2, body, 0)
        scatter_wait(0)
        scatter_wait(1)
        base = s * ES + NG_FULL * G
        pltpu.sync_copy(dst_hbm.at[pl.ds(base, TAIL)],
                        dtmp.at[0, pl.ds(0, TAIL)])
        _remap_dst(dtmp.at[0], ldst_t.at[0], c, TAIL // 16)
        pltpu.sync_copy(ones_v.at[pl.ds(0, TAIL)],
                        acc_sh.at[ldst_t.at[0]], add=True)
        plsc.subcore_barrier()
        _dump_half(acc_sh, out_hbm, c, s)
        plsc.subcore_barrier()


_cnt = functools.partial(
    pl.kernel,
    out_type=(jax.ShapeDtypeStruct((NC, HPAD, H), jnp.float32),
              jax.ShapeDtypeStruct((NC, HPAD, H), jnp.float32)),
    mesh=_mesh,
    compiler_params=_SC_PARAMS,
    scratch_types=[
        pltpu.VMEM_SHARED((HPAD, H), jnp.float32),
        pltpu.VMEM((1, G), jnp.int32),
        pltpu.VMEM((2, G), jnp.int32),
        pltpu.VMEM((1, TAIL), jnp.int32),
        pltpu.VMEM((G, H), jnp.float32),
        pltpu.SemaphoreType.DMA,
        pltpu.SemaphoreType.DMA,
    ],
)(_cnt_body)


GB_FULL = B // G
GTAIL = B - GB_FULL * G
GPW = GB_FULL // 32
GREM = GB_FULL - GPW * 32


def _fgather_body(ui_hbm, ii_hbm, utab_hbm, itab_hbm, uh_hbm, ih_hbm,
                  eidx, rows,
                  gsem0, gsem1, gsem2, gsem3, osem0, osem1, osem2, osem3):
    c = lax.axis_index("c")
    s = lax.axis_index("s")
    w = s * NC + c
    gsems = (gsem0, gsem1, gsem2, gsem3)
    osems = (osem0, osem1, osem2, osem3)
    srcs = ((ui_hbm, utab_hbm, uh_hbm), (ii_hbm, itab_hbm, ih_hbm))

    def load_gather(k, d, bi):
        idx_hbm, tab_hbm, _ = srcs[d]
        base = (k * 32 + w) * G
        pltpu.sync_copy(idx_hbm.at[pl.ds(base, G)], eidx.at[bi])
        _remap_src(eidx.at[bi], G // 16)
        pltpu.async_copy(tab_hbm.at[eidx.at[bi]], rows.at[bi], gsems[bi])

    def gather_wait(d, bi):
        pltpu.make_async_copy(srcs[d][1].at[eidx.at[bi]], rows.at[bi],
                              gsems[bi]).wait()

    def write_start(k, d, bi):
        base = (k * 32 + w) * G
        pltpu.async_copy(rows.at[bi], srcs[d][2].at[pl.ds(base, G)],
                         osems[bi])

    def write_wait(k, d, bi):
        base = (k * 32 + w) * G
        pltpu.make_async_copy(rows.at[bi], srcs[d][2].at[pl.ds(base, G)],
                              osems[bi]).wait()

    def body(k0, carry):
        for kk in (0, 1):
            k = k0 * 2 + kk
            for d in (0, 1):
                bi = 2 * d + kk
                ob = 2 * d + (1 - kk)

                @pl.when(k0 >= 1)
                def _():
                    write_wait(k - 2, d, bi)

                load_gather(k, d, bi)

                if kk == 0:
                    @pl.when(k0 >= 1)
                    def _():
                        gather_wait(d, ob)
                        write_start(k - 1, d, ob)
                else:
                    gather_wait(d, ob)
                    write_start(k - 1, d, ob)
        return carry

    lax.fori_loop(0, GPW // 2, body, 0)
    for d in (0, 1):
        gather_wait(d, 2 * d + 1)
        write_start(GPW - 1, d, 2 * d + 1)
    for d in (0, 1):
        write_wait(GPW - 2, d, 2 * d)
        write_wait(GPW - 1, d, 2 * d + 1)

    def do_pair(base, gcount, n16):
        for d in (0, 1):
            idx_hbm, tab_hbm, out_hbm = srcs[d]
            pltpu.sync_copy(idx_hbm.at[pl.ds(base, gcount)],
                            eidx.at[0, pl.ds(0, gcount)])
            _remap_src(eidx.at[0], n16)
            pltpu.async_copy(tab_hbm.at[eidx.at[0, pl.ds(0, gcount)]],
                             rows.at[0, pl.ds(0, gcount)], gsem0).wait()
            pltpu.sync_copy(rows.at[0, pl.ds(0, gcount)],
                            out_hbm.at[pl.ds(base, gcount)])

    @pl.when(w < GREM)
    def _():
        do_pair((GPW * 32 + w) * G, G, G // 16)

    @pl.when(w == 31)
    def _():
        do_pair(GB_FULL * G, GTAIL, GTAIL // 16)


_fgather = functools.partial(
    pl.kernel,
    out_type=(jax.ShapeDtypeStruct((B, H), jnp.float32),
              jax.ShapeDtypeStruct((B, H), jnp.float32)),
    mesh=_mesh,
    compiler_params=_SC_PARAMS,
    scratch_types=[
        pltpu.VMEM((4, G), jnp.int32),
        pltpu.VMEM((4, G, H), jnp.float32),
    ] + [pltpu.SemaphoreType.DMA] * 8,
)(_fgather_body)


DBLK = 512
MBLK = 1000


def _dense_kernel(acc_ref, cnt_ref, x_ref, a_ref, c_ref, d_ref, o_ref):
    mean = acc_ref[...] / jnp.maximum(cnt_ref[...], 1.0)
    y = (jnp.dot(mean, a_ref[...], preferred_element_type=jnp.float32)
         + jnp.dot(x_ref[...], c_ref[...], preferred_element_type=jnp.float32)
         + d_ref[...])
    o_ref[...] = jnp.maximum(y, 0.0)


def _dense(acc, cnt, x, a, cm, d):
    grid = (NPAD // DBLK,)
    blk = pl.BlockSpec((DBLK, H), lambda i: (i, 0))
    wblk = pl.BlockSpec((H, H), lambda i: (0, 0))
    return pl.pallas_call(
        _dense_kernel,
        grid=grid,
        in_specs=[blk, blk, blk, wblk, wblk, pl.BlockSpec((1, H), lambda i: (0, 0))],
        out_specs=blk,
        out_shape=jax.ShapeDtypeStruct((NPAD, H), jnp.float32),
    )(acc, cnt, x, a, cm, d)


def _mlp_kernel(u_ref, i_ref, w1u_ref, w1i_ref, b1_ref, w2_ref, b2_ref, o_ref):
    h = (jnp.dot(u_ref[...], w1u_ref[...], preferred_element_type=jnp.float32)
         + jnp.dot(i_ref[...], w1i_ref[...], preferred_element_type=jnp.float32)
         + b1_ref[...])
    h = jnp.maximum(h, 0.0)
    o_ref[...] = (jnp.dot(h, w2_ref[...], preferred_element_type=jnp.float32)
                  + b2_ref[...])


def _mlp(uh, ih, w1u, w1i, b1, w2t, b2):
    grid = (B // MBLK,)
    blk = pl.BlockSpec((MBLK, H), lambda i: (i, 0))
    return pl.pallas_call(
        _mlp_kernel,
        grid=grid,
        in_specs=[blk, blk,
                  pl.BlockSpec((H, H), lambda i: (0, 0)),
                  pl.BlockSpec((H, H), lambda i: (0, 0)),
                  pl.BlockSpec((1, H), lambda i: (0, 0)),
                  pl.BlockSpec((H, 8), lambda i: (0, 0)),
                  pl.BlockSpec((1, 8), lambda i: (0, 0))],
        out_specs=pl.BlockSpec((MBLK, 8), lambda i: (i, 0)),
        out_shape=jax.ShapeDtypeStruct((B, 8), jnp.float32),
    )(uh, ih, w1u, w1i, b1, w2t, b2)


def _pad_table(x):
    z = jnp.zeros((PADW, H), jnp.float32)
    return jnp.concatenate([x[:HALF], z, x[HALF:], z], axis=0)


def kernel(edge_index, edge_label_index, ue, ie, uWl, uWr, ub, ug, ubb,
           iWl, iWr, ib, ig, ibb, fcW1, fcb1, fcW2, fcb2):
    u_idx = edge_index[0].astype(jnp.int32)
    i_idx = edge_index[1].astype(jnp.int32)
    eli_u = edge_label_index[0].astype(jnp.int32)
    eli_i = edge_label_index[1].astype(jnp.int32)

    zeros = jnp.zeros((ZS, H), jnp.float32)
    ones = jnp.ones((G, H), jnp.float32)

    upad = _pad_table(ue)
    ipad = _pad_table(ie)

    cntu, cnti = _cnt(u_idx, i_idx, ones, zeros)
    cntu = cntu.reshape(NPAD, H)
    cnti = cnti.reshape(NPAD, H)

    gscale = 1.0 / jnp.sqrt(1.0 + EPS)
    for l in range(NLAYERS):
        accu = _agg(i_idx, u_idx, ipad, zeros).reshape(NPAD, H)
        acci = _agg(u_idx, i_idx, upad, zeros).reshape(NPAD, H)
        gu = ug[l] * gscale
        gi = ig[l] * gscale
        au = uWl[l].T * gu[None, :]
        cu = uWr[l].T * gu[None, :]
        du = (ub[l] * gu + ubb[l])[None, :]
        ai = iWl[l].T * gi[None, :]
        ci = iWr[l].T * gi[None, :]
        di = (ib[l] * gi + ibb[l])[None, :]
        new_u = _dense(accu, cntu, upad, au, cu, du)
        new_i = _dense(acci, cnti, ipad, ai, ci, di)
        upad, ipad = new_u, new_i

    uh, ih = _fgather(eli_u, eli_i, upad, ipad)

    w1u = fcW1[:, :H].T
    w1i = fcW1[:, H:].T
    b1 = fcb1[None, :]
    w2t = jnp.zeros((H, 8), jnp.float32).at[:, :4].set(fcW2.T)
    b2 = jnp.zeros((1, 8), jnp.float32).at[0, :4].set(fcb2)
    out8 = _mlp(uh, ih, w1u, w1i, b1, w2t, b2)
    return out8[:, :4]


# speedup vs baseline: 3.3146x; 1.0104x over previous
import functools

import jax
import jax.numpy as jnp
from jax import lax
from jax.experimental import pallas as pl
from jax.experimental.pallas import tpu as pltpu
from jax.experimental.pallas import tpu_sc as plsc

N = 50000
H = 64
E = 800000
B = 200000
NLAYERS = 2
EPS = 1e-5

HALF = 25000
TRASH = HALF
HPAD = 25088
NPAD = 2 * HPAD
PADW = HPAD - HALF

NC = 2
NS = 16
ES = E // NS
G = 128
NG_FULL = ES // G
TAIL = ES - NG_FULL * G
ZS = HPAD // NS
CW = 16

_mesh = plsc.VectorSubcoreMesh(core_axis_name="c", subcore_axis_name="s")


def _remap_src(idx_ref, n16):
    for i in range(n16):
        v = idx_ref[pl.ds(i * 16, 16)]
        idx_ref[pl.ds(i * 16, 16)] = jnp.where(v >= HALF, v + PADW, v)


def _remap_dst(raw_ref, out_ref, c, n16):
    base = c * HALF
    for i in range(n16):
        v = raw_ref[pl.ds(i * 16, 16)] - base
        ok = (v >= 0) & (v < HALF)
        out_ref[pl.ds(i * 16, 16)] = jnp.where(ok, v, TRASH)


def _zero_and_barrier(zeros_hbm, acc_sh, s):
    pltpu.sync_copy(zeros_hbm, acc_sh.at[pl.ds(s * ZS, ZS)])
    plsc.subcore_barrier()


def _dump_half(acc_sh, out_hbm, c, s):
    pltpu.sync_copy(acc_sh.at[pl.ds(s * ZS, ZS)],
                    out_hbm.at[c, pl.ds(s * ZS, ZS)])


def _agg_body(src_hbm, dst_hbm, table_hbm, zeros_hbm, out_hbm,
              acc_sh, sidx, dtmp, ldst, sidx_t, ldst_t, rows,
              gsem0, gsem1, ssem0, ssem1):
    c = lax.axis_index("c")
    s = lax.axis_index("s")
    _zero_and_barrier(zeros_hbm, acc_sh, s)
    gsems = (gsem0, gsem1)
    ssems = (ssem0, ssem1)

    def load_group(g, b):
        base = s * ES + g * G
        pltpu.sync_copy(src_hbm.at[pl.ds(base, G)], sidx.at[b])
        pltpu.sync_copy(dst_hbm.at[pl.ds(base, G)], dtmp.at[0])
        _remap_src(sidx.at[b], G // 16)
        _remap_dst(dtmp.at[0], ldst.at[b], c, G // 16)

    def gather_start(b):
        pltpu.async_copy(table_hbm.at[sidx.at[b]], rows.at[b], gsems[b])

    def gather_wait(b):
        pltpu.make_async_copy(table_hbm.at[sidx.at[b]], rows.at[b],
                              gsems[b]).wait()

    def scatter_start(b):
        pltpu.async_copy(rows.at[b], acc_sh.at[ldst.at[b]], ssems[b],
                         add=True)

    def scatter_wait(b):
        pltpu.make_async_copy(rows.at[b], acc_sh.at[ldst.at[b]],
                              ssems[b]).wait()

    def body(g0, carry):
        for b in (0, 1):
            g = g0 * 2 + b

            @pl.when(g0 >= 1)
            def _():
                scatter_wait(b)

            load_group(g, b)
            gather_start(b)

            if b == 0:
                @pl.when(g0 >= 1)
                def _():
                    gather_wait(1)
                    scatter_start(1)
            else:
                gather_wait(0)
                scatter_start(0)
        return carry

    lax.fori_loop(0, NG_FULL // 2, body, 0)
    gather_wait(1)
    scatter_start(1)
    scatter_wait(0)
    base = s * ES + NG_FULL * G
    pltpu.sync_copy(src_hbm.at[pl.ds(base, TAIL)], sidx_t.at[0])
    pltpu.sync_copy(dst_hbm.at[pl.ds(base, TAIL)], dtmp.at[0, pl.ds(0, TAIL)])
    _remap_src(sidx_t.at[0], TAIL // 16)
    _remap_dst(dtmp.at[0], ldst_t.at[0], c, TAIL // 16)
    pltpu.async_copy(table_hbm.at[sidx_t.at[0]],
                     rows.at[0, pl.ds(0, TAIL)], gsem0).wait()
    scatter_wait(1)
    pltpu.sync_copy(rows.at[0, pl.ds(0, TAIL)],
                    acc_sh.at[ldst_t.at[0]], add=True)
    plsc.subcore_barrier()
    _dump_half(acc_sh, out_hbm, c, s)


_SC_PARAMS = pltpu.CompilerParams(use_tc_tiling_on_sc=False)

_agg = functools.partial(
    pl.kernel,
    out_type=jax.ShapeDtypeStruct((NC, HPAD, H), jnp.float32),
    mesh=_mesh,
    compiler_params=_SC_PARAMS,
    scratch_types=[
        pltpu.VMEM_SHARED((HPAD, H), jnp.float32),
        pltpu.VMEM((2, G), jnp.int32),
        pltpu.VMEM((1, G), jnp.int32),
        pltpu.VMEM((2, G), jnp.int32),
        pltpu.VMEM((1, TAIL), jnp.int32),
        pltpu.VMEM((1, TAIL), jnp.int32),
        pltpu.VMEM((2, G, H), jnp.float32),
        pltpu.SemaphoreType.DMA,
        pltpu.SemaphoreType.DMA,
        pltpu.SemaphoreType.DMA,
        pltpu.SemaphoreType.DMA,
    ],
)(_agg_body)


def _cnt_body(udst_hbm, idst_hbm, ones_hbm, zeros_hbm, outu_hbm, outi_hbm,
              acc_sh, dtmp, ldst, ldst_t, ones_v, ssem0, ssem1):
    c = lax.axis_index("c")
    s = lax.axis_index("s")
    pltpu.sync_copy(ones_hbm, ones_v)
    ssems = (ssem0, ssem1)

    for dst_hbm, out_hbm in ((udst_hbm, outu_hbm), (idst_hbm, outi_hbm)):
        _zero_and_barrier(zeros_hbm, acc_sh, s)

        def scatter_wait(b):
            pltpu.make_async_copy(ones_v, acc_sh.at[ldst.at[b]],
                                  ssems[b]).wait()

        def body(g0, carry):
            for b in (0, 1):
                g = g0 * 2 + b

                @pl.when(g0 >= 1)
                def _():
                    scatter_wait(b)

                base = s * ES + g * G
                pltpu.sync_copy(dst_hbm.at[pl.ds(base, G)],
                                dtmp.at[0])
                _remap_dst(dtmp.at[0], ldst.at[b], c, G // 16)
                pltpu.async_copy(ones_v, acc_sh.at[ldst.at[b]], ssems[b],
                                 add=True)
            return carry

        lax.fori_loop(0, NG_FULL // 2, body, 0)
        scatter_wait(0)
        scatter_wait(1)
        base = s * ES + NG_FULL * G
        pltpu.sync_copy(dst_hbm.at[pl.ds(base, TAIL)],
                        dtmp.at[0, pl.ds(0, TAIL)])
        _remap_dst(dtmp.at[0], ldst_t.at[0], c, TAIL // 16)
        pltpu.sync_copy(ones_v.at[pl.ds(0, TAIL)],
                        acc_sh.at[ldst_t.at[0]], add=True)
        plsc.subcore_barrier()
        _dump_half(acc_sh, out_hbm, c, s)
        plsc.subcore_barrier()


_cnt = functools.partial(
    pl.kernel,
    out_type=(jax.ShapeDtypeStruct((NC, HPAD, CW), jnp.float32),
              jax.ShapeDtypeStruct((NC, HPAD, CW), jnp.float32)),
    mesh=_mesh,
    compiler_params=_SC_PARAMS,
    scratch_types=[
        pltpu.VMEM_SHARED((HPAD, CW), jnp.float32),
        pltpu.VMEM((1, G), jnp.int32),
        pltpu.VMEM((2, G), jnp.int32),
        pltpu.VMEM((1, TAIL), jnp.int32),
        pltpu.VMEM((G, CW), jnp.float32),
        pltpu.SemaphoreType.DMA,
        pltpu.SemaphoreType.DMA,
    ],
)(_cnt_body)


GB_FULL = B // G
GTAIL = B - GB_FULL * G
GPW = GB_FULL // 32
GREM = GB_FULL - GPW * 32


def _fgather_body(ui_hbm, ii_hbm, utab_hbm, itab_hbm, uh_hbm, ih_hbm,
                  eidx, rows,
                  gsem0, gsem1, gsem2, gsem3, osem0, osem1, osem2, osem3):
    c = lax.axis_index("c")
    s = lax.axis_index("s")
    w = s * NC + c
    gsems = (gsem0, gsem1, gsem2, gsem3)
    osems = (osem0, osem1, osem2, osem3)
    srcs = ((ui_hbm, utab_hbm, uh_hbm), (ii_hbm, itab_hbm, ih_hbm))

    def load_gather(k, d, bi):
        idx_hbm, tab_hbm, _ = srcs[d]
        base = (k * 32 + w) * G
        pltpu.sync_copy(idx_hbm.at[pl.ds(base, G)], eidx.at[bi])
        _remap_src(eidx.at[bi], G // 16)
        pltpu.async_copy(tab_hbm.at[eidx.at[bi]], rows.at[bi], gsems[bi])

    def gather_wait(d, bi):
        pltpu.make_async_copy(srcs[d][1].at[eidx.at[bi]], rows.at[bi],
                              gsems[bi]).wait()

    def write_start(k, d, bi):
        base = (k * 32 + w) * G
        pltpu.async_copy(rows.at[bi], srcs[d][2].at[pl.ds(base, G)],
                         osems[bi])

    def write_wait(k, d, bi):
        base = (k * 32 + w) * G
        pltpu.make_async_copy(rows.at[bi], srcs[d][2].at[pl.ds(base, G)],
                              osems[bi]).wait()

    def body(k0, carry):
        for kk in (0, 1):
            k = k0 * 2 + kk
            for d in (0, 1):
                bi = 2 * d + kk
                ob = 2 * d + (1 - kk)

                @pl.when(k0 >= 1)
                def _():
                    write_wait(k - 2, d, bi)

                load_gather(k, d, bi)

                if kk == 0:
                    @pl.when(k0 >= 1)
                    def _():
                        gather_wait(d, ob)
                        write_start(k - 1, d, ob)
                else:
                    gather_wait(d, ob)
                    write_start(k - 1, d, ob)
        return carry

    lax.fori_loop(0, GPW // 2, body, 0)
    for d in (0, 1):
        gather_wait(d, 2 * d + 1)
        write_start(GPW - 1, d, 2 * d + 1)
    for d in (0, 1):
        write_wait(GPW - 2, d, 2 * d)
        write_wait(GPW - 1, d, 2 * d + 1)

    def do_pair(base, gcount, n16):
        for d in (0, 1):
            idx_hbm, tab_hbm, out_hbm = srcs[d]
            pltpu.sync_copy(idx_hbm.at[pl.ds(base, gcount)],
                            eidx.at[0, pl.ds(0, gcount)])
            _remap_src(eidx.at[0], n16)
            pltpu.async_copy(tab_hbm.at[eidx.at[0, pl.ds(0, gcount)]],
                             rows.at[0, pl.ds(0, gcount)], gsem0).wait()
            pltpu.sync_copy(rows.at[0, pl.ds(0, gcount)],
                            out_hbm.at[pl.ds(base, gcount)])

    @pl.when(w < GREM)
    def _():
        do_pair((GPW * 32 + w) * G, G, G // 16)

    @pl.when(w == 31)
    def _():
        do_pair(GB_FULL * G, GTAIL, GTAIL // 16)


_fgather = functools.partial(
    pl.kernel,
    out_type=(jax.ShapeDtypeStruct((B, H), jnp.float32),
              jax.ShapeDtypeStruct((B, H), jnp.float32)),
    mesh=_mesh,
    compiler_params=_SC_PARAMS,
    scratch_types=[
        pltpu.VMEM((4, G), jnp.int32),
        pltpu.VMEM((4, G, H), jnp.float32),
    ] + [pltpu.SemaphoreType.DMA] * 8,
)(_fgather_body)


DBLK = 512
MBLK = 1000


def _dense_kernel(acc_ref, cnt_ref, x_ref, a_ref, c_ref, d_ref, o_ref):
    mean = acc_ref[...] / jnp.maximum(cnt_ref[:, :1], 1.0)
    y = (jnp.dot(mean, a_ref[...], preferred_element_type=jnp.float32)
         + jnp.dot(x_ref[...], c_ref[...], preferred_element_type=jnp.float32)
         + d_ref[...])
    o_ref[...] = jnp.maximum(y, 0.0)


def _dense(acc, cnt, x, a, cm, d):
    grid = (NPAD // DBLK,)
    blk = pl.BlockSpec((DBLK, H), lambda i: (i, 0))
    cblk = pl.BlockSpec((DBLK, CW), lambda i: (i, 0))
    wblk = pl.BlockSpec((H, H), lambda i: (0, 0))
    return pl.pallas_call(
        _dense_kernel,
        grid=grid,
        in_specs=[blk, cblk, blk, wblk, wblk, pl.BlockSpec((1, H), lambda i: (0, 0))],
        out_specs=blk,
        out_shape=jax.ShapeDtypeStruct((NPAD, H), jnp.float32),
    )(acc, cnt, x, a, cm, d)


def _mlp_kernel(u_ref, i_ref, w1u_ref, w1i_ref, b1_ref, w2_ref, b2_ref, o_ref):
    h = (jnp.dot(u_ref[...], w1u_ref[...], preferred_element_type=jnp.float32)
         + jnp.dot(i_ref[...], w1i_ref[...], preferred_element_type=jnp.float32)
         + b1_ref[...])
    h = jnp.maximum(h, 0.0)
    o_ref[...] = (jnp.dot(h, w2_ref[...], preferred_element_type=jnp.float32)
                  + b2_ref[...])


def _mlp(uh, ih, w1u, w1i, b1, w2t, b2):
    grid = (B // MBLK,)
    blk = pl.BlockSpec((MBLK, H), lambda i: (i, 0))
    return pl.pallas_call(
        _mlp_kernel,
        grid=grid,
        in_specs=[blk, blk,
                  pl.BlockSpec((H, H), lambda i: (0, 0)),
                  pl.BlockSpec((H, H), lambda i: (0, 0)),
                  pl.BlockSpec((1, H), lambda i: (0, 0)),
                  pl.BlockSpec((H, 8), lambda i: (0, 0)),
                  pl.BlockSpec((1, 8), lambda i: (0, 0))],
        out_specs=pl.BlockSpec((MBLK, 8), lambda i: (i, 0)),
        out_shape=jax.ShapeDtypeStruct((B, 8), jnp.float32),
    )(uh, ih, w1u, w1i, b1, w2t, b2)


def _pad_table(x):
    z = jnp.zeros((PADW, H), jnp.float32)
    return jnp.concatenate([x[:HALF], z, x[HALF:], z], axis=0)


def kernel(edge_index, edge_label_index, ue, ie, uWl, uWr, ub, ug, ubb,
           iWl, iWr, ib, ig, ibb, fcW1, fcb1, fcW2, fcb2):
    u_idx = edge_index[0].astype(jnp.int32)
    i_idx = edge_index[1].astype(jnp.int32)
    eli_u = edge_label_index[0].astype(jnp.int32)
    eli_i = edge_label_index[1].astype(jnp.int32)

    zeros = jnp.zeros((ZS, H), jnp.float32)
    zeros16 = jnp.zeros((ZS, CW), jnp.float32)
    ones = jnp.ones((G, CW), jnp.float32)

    upad = _pad_table(ue)
    ipad = _pad_table(ie)

    cntu, cnti = _cnt(u_idx, i_idx, ones, zeros16)
    cntu = cntu.reshape(NPAD, CW)
    cnti = cnti.reshape(NPAD, CW)

    gscale = 1.0 / jnp.sqrt(1.0 + EPS)
    for l in range(NLAYERS):
        accu = _agg(i_idx, u_idx, ipad, zeros).reshape(NPAD, H)
        acci = _agg(u_idx, i_idx, upad, zeros).reshape(NPAD, H)
        gu = ug[l] * gscale
        gi = ig[l] * gscale
        au = uWl[l].T * gu[None, :]
        cu = uWr[l].T * gu[None, :]
        du = (ub[l] * gu + ubb[l])[None, :]
        ai = iWl[l].T * gi[None, :]
        ci = iWr[l].T * gi[None, :]
        di = (ib[l] * gi + ibb[l])[None, :]
        new_u = _dense(accu, cntu, upad, au, cu, du)
        new_i = _dense(acci, cnti, ipad, ai, ci, di)
        upad, ipad = new_u, new_i

    uh, ih = _fgather(eli_u, eli_i, upad, ipad)

    w1u = fcW1[:, :H].T
    w1i = fcW1[:, H:].T
    b1 = fcb1[None, :]
    w2t = jnp.zeros((H, 8), jnp.float32).at[:, :4].set(fcW2.T)
    b2 = jnp.zeros((1, 8), jnp.float32).at[0, :4].set(fcb2)
    out8 = _mlp(uh, ih, w1u, w1i, b1, w2t, b2)
    return out8[:, :4]
